# baseline (speedup 1.0000x reference)
import jax
import jax.numpy as jnp
from jax import lax
from jax.experimental import pallas as pl
from jax.experimental.pallas import tpu as pltpu
from jax.experimental.pallas import tpu_sc as plsc

EMBED = 64
EPS = 1e-5

_INFO = plsc.get_sparse_core_info()
NC = _INFO.num_cores
NS = _INFO.num_subcores
L = _INFO.num_lanes
NW = NC * NS

CHUNK = 512
NK = EMBED // L


def _rsqrt(x):
    i = plsc.bitcast(x, jnp.int32)
    i = jnp.int32(0x5F3759DF) - lax.shift_right_logical(i, 1)
    y = plsc.bitcast(i, jnp.float32)
    for _ in range(3):
        y = y * (1.5 - 0.5 * x * y * y)
    return y


def _ln_body(ids_hbm, table_hbm, gamma_hbm, beta_hbm, out_hbm,
             idx_v, rows_v, gb_v, sem):
    n_per_w = ids_hbm.shape[0] // NW
    wid = lax.axis_index("s") * NC + lax.axis_index("c")
    base_w = wid * n_per_w

    pltpu.sync_copy(gamma_hbm, gb_v.at[0])
    pltpu.sync_copy(beta_hbm, gb_v.at[1])
    gamma_k = [gb_v[0, pl.ds(k * L, L)] for k in range(NK)]
    beta_k = [gb_v[1, pl.ds(k * L, L)] for k in range(NK)]

    def chunk_body(n, _):
        base = base_w + n * CHUNK
        pltpu.sync_copy(ids_hbm.at[pl.ds(base, CHUNK)], idx_v)
        pltpu.async_copy(table_hbm.at[idx_v], rows_v, sem).wait()

        def row_body(r, _):
            x = [rows_v[r, pl.ds(k * L, L)] for k in range(NK)]
            s = x[0] + x[1] + x[2] + x[3]
            sq = x[0] * x[0] + x[1] * x[1] + x[2] * x[2] + x[3] * x[3]
            mean_s = jnp.sum(s) * (1.0 / EMBED)
            var_s = jnp.sum(sq) * (1.0 / EMBED) - mean_s * mean_s
            rstd = _rsqrt(jnp.full((L,), var_s + EPS, jnp.float32))
            m = jnp.full((L,), mean_s, jnp.float32)
            for k in range(NK):
                rows_v[r, pl.ds(k * L, L)] = (
                    (x[k] - m) * rstd * gamma_k[k] + beta_k[k]
                )
            return ()

        lax.fori_loop(0, CHUNK, row_body, (), unroll=False)
        pltpu.sync_copy(rows_v, out_hbm.at[pl.ds(base, CHUNK)])
        return ()

    lax.fori_loop(0, n_per_w // CHUNK, chunk_body, (), unroll=False)


def kernel(input_ids, table, gamma, beta):
    batch, seq = input_ids.shape
    n = batch * seq
    ids_flat = input_ids.reshape(n).astype(jnp.int32)

    mesh = plsc.VectorSubcoreMesh(core_axis_name="c", subcore_axis_name="s")
    run = pl.kernel(
        _ln_body,
        out_type=jax.ShapeDtypeStruct((n, EMBED), jnp.float32),
        mesh=mesh,
        scratch_types=[
            pltpu.VMEM((CHUNK,), jnp.int32),
            pltpu.VMEM((CHUNK, EMBED), jnp.float32),
            pltpu.VMEM((2, EMBED), jnp.float32),
            pltpu.SemaphoreType.DMA,
        ],
        compiler_params=pltpu.CompilerParams(
            needs_layout_passes=False, use_tc_tiling_on_sc=False
        ),
    )
    out = run(ids_flat, table, gamma, beta)
    return out.reshape(batch, seq, EMBED)
